# baseline (speedup 1.0000x reference)
import functools

import jax
import jax.numpy as jnp
from jax import lax
from jax.experimental import pallas as pl
from jax.experimental.pallas import tpu as pltpu
from jax.experimental.pallas import tpu_sc as plsc

_N, _D, _S, _E = 8192, 1024, 512, 8
_B = 128
_M = _N + _E * _B
_G = _M // _B
_NW = 32
_ZW = 128


def _sc_worker_id():
    return lax.axis_index("s") * 2 + lax.axis_index("c")


@functools.lru_cache(maxsize=None)
def _make_scatter_x_z(chunk):
    rows_per_w = _N // _NW
    n_chunks = rows_per_w // chunk
    mesh = plsc.VectorSubcoreMesh(core_axis_name="c", subcore_axis_name="s")

    @functools.partial(
        pl.kernel,
        mesh=mesh,
        out_type=[
            jax.ShapeDtypeStruct((_M, _D), jnp.float32),
            jax.ShapeDtypeStruct((_M, _ZW), jnp.float32),
        ],
        scratch_types=[
            pltpu.VMEM((chunk,), jnp.int32),
            pltpu.VMEM((chunk, _D), jnp.float32),
            pltpu.VMEM((chunk, _ZW), jnp.float32),
            pltpu.SemaphoreType.DMA,
            pltpu.SemaphoreType.DMA,
        ],
    )
    def scatter(x_hbm, zw_hbm, dest_hbm, xg_hbm, zg_hbm,
                idx_v, rows_v, zrows_v, sem, zsem):
        base = _sc_worker_id() * rows_per_w

        def body(i, carry):
            off = pl.multiple_of(base + i * chunk, 8)
            pltpu.sync_copy(dest_hbm.at[pl.ds(off, chunk)], idx_v)
            pltpu.sync_copy(x_hbm.at[pl.ds(off, chunk)], rows_v)
            pltpu.sync_copy(zw_hbm.at[pl.ds(off, chunk)], zrows_v)
            cp = pltpu.async_copy(rows_v, xg_hbm.at[idx_v], sem)
            zcp = pltpu.async_copy(zrows_v, zg_hbm.at[idx_v], zsem)
            cp.wait()
            zcp.wait()
            return carry

        lax.fori_loop(0, n_chunks, body, 0)

    return scatter


@functools.lru_cache(maxsize=None)
def _make_gather_rows(n_rows_out, chunk):
    rows_per_w = n_rows_out // _NW
    n_chunks = rows_per_w // chunk
    mesh = plsc.VectorSubcoreMesh(core_axis_name="c", subcore_axis_name="s")

    @functools.partial(
        pl.kernel,
        mesh=mesh,
        out_type=jax.ShapeDtypeStruct((n_rows_out, _D), jnp.float32),
        scratch_types=[
            pltpu.VMEM((chunk,), jnp.int32),
            pltpu.VMEM((chunk, _D), jnp.float32),
            pltpu.SemaphoreType.DMA,
        ],
    )
    def gather(table_hbm, idx_hbm, out_hbm, idx_v, rows_v, sem):
        base = _sc_worker_id() * rows_per_w

        def body(i, carry):
            off = pl.multiple_of(base + i * chunk, 8)
            pltpu.sync_copy(idx_hbm.at[pl.ds(off, chunk)], idx_v)
            pltpu.async_copy(table_hbm.at[idx_v], rows_v, sem).wait()
            pltpu.sync_copy(rows_v, out_hbm.at[pl.ds(off, chunk)])
            return carry

        lax.fori_loop(0, n_chunks, body, 0)

    return gather


def _moe_block_body(be_ref, xg_ref, zg_ref, w1_ref, b1_ref, w2_ref, b2_ref,
                    out_ref):
    xr = xg_ref[...]
    a = jnp.maximum(xr, 0.0).astype(jnp.bfloat16)
    h = lax.dot_general(a, w1_ref[0], (((1,), (1,)), ((), ())),
                        preferred_element_type=jnp.float32)
    h = jnp.maximum(h + b1_ref[0, 0, :][None, :], 0.0).astype(jnp.bfloat16)
    o = lax.dot_general(h, w2_ref[0], (((1,), (1,)), ((), ())),
                        preferred_element_type=jnp.float32)
    o = o + b2_ref[0, 0, :][None, :]
    out_ref[...] = xr + zg_ref[:, 0:1] * o


def _grouped_fc(be, xg, zg, W1, b1, W2, b2):
    grid_spec = pltpu.PrefetchScalarGridSpec(
        num_scalar_prefetch=1,
        grid=(_G,),
        in_specs=[
            pl.BlockSpec((_B, _D), lambda g, be_r: (g, 0)),
            pl.BlockSpec((_B, _ZW), lambda g, be_r: (g, 0)),
            pl.BlockSpec((1, _S, _D), lambda g, be_r: (be_r[g], 0, 0)),
            pl.BlockSpec((1, 1, _S), lambda g, be_r: (be_r[g], 0, 0)),
            pl.BlockSpec((1, _D, _S), lambda g, be_r: (be_r[g], 0, 0)),
            pl.BlockSpec((1, 1, _D), lambda g, be_r: (be_r[g], 0, 0)),
        ],
        out_specs=pl.BlockSpec((_B, _D), lambda g, be_r: (g, 0)),
    )
    return pl.pallas_call(
        _moe_block_body,
        grid_spec=grid_spec,
        out_shape=jax.ShapeDtypeStruct((_M, _D), jnp.float32),
    )(be, xg, zg, W1.astype(jnp.bfloat16), b1.reshape(_E, 1, _S),
      W2.astype(jnp.bfloat16), b2.reshape(_E, 1, _D))


def _scatter_x_z(x, z, dest):
    zwide = jnp.broadcast_to(z, (_N, _ZW))
    return _make_scatter_x_z(64)(x, zwide, dest)


def _gather_out(outg, dest):
    return _make_gather_rows(_N, 64)(outg, dest)


def _routing(yi):
    onehot = (yi[:, None] == jnp.arange(_E, dtype=jnp.int32)[None, :])
    rank_incl = jnp.cumsum(onehot.astype(jnp.int32), axis=0)
    counts = rank_incl[-1]
    rank = jnp.sum(rank_incl * onehot, axis=1) - 1
    nblk = (counts + _B - 1) // _B
    blk_cum = jnp.cumsum(nblk)
    slot_start = (blk_cum - nblk) * _B
    dest = slot_start[yi] + rank
    g_idx = jnp.arange(_G, dtype=jnp.int32)
    be = jnp.sum(
        (g_idx[:, None] >= blk_cum[None, :]).astype(jnp.int32), axis=1)
    be = jnp.minimum(be, _E - 1)
    return dest, be


def kernel(x, y_index, y_hard, z, W1, b1, W2, b2):
    del y_hard
    yi = y_index[:, 0].astype(jnp.int32)
    dest, be = _routing(yi)
    xg, zg = _scatter_x_z(x, z, dest)
    outg = _grouped_fc(be, xg, zg, W1, b1, W2, b2)
    return _gather_out(outg, dest)


# speedup vs baseline: 1.2419x; 1.2419x over previous
import functools

import jax
import jax.numpy as jnp
from jax import lax
from jax.experimental import pallas as pl
from jax.experimental.pallas import tpu as pltpu
from jax.experimental.pallas import tpu_sc as plsc

_N, _D, _S, _E = 8192, 1024, 512, 8
_B = 256
_M = _N + _E * _B
_G = _M // _B
_NW = 32
_ZW = 128


def _sc_worker_id():
    return lax.axis_index("s") * 2 + lax.axis_index("c")


@functools.lru_cache(maxsize=None)
def _make_scatter_x_z(chunk):
    rows_per_w = _N // _NW
    n_chunks = rows_per_w // chunk
    mesh = plsc.VectorSubcoreMesh(core_axis_name="c", subcore_axis_name="s")

    @functools.partial(
        pl.kernel,
        mesh=mesh,
        out_type=[
            jax.ShapeDtypeStruct((_M, _D), jnp.float32),
            jax.ShapeDtypeStruct((_M, _ZW), jnp.float32),
        ],
        scratch_types=[
            pltpu.VMEM((chunk,), jnp.int32),
            pltpu.VMEM((chunk, _D), jnp.float32),
            pltpu.VMEM((chunk, _ZW), jnp.float32),
            pltpu.SemaphoreType.DMA,
            pltpu.SemaphoreType.DMA,
        ],
    )
    def scatter(x_hbm, zw_hbm, dest_hbm, xg_hbm, zg_hbm,
                idx_v, rows_v, zrows_v, sem, zsem):
        base = _sc_worker_id() * rows_per_w

        def body(i, carry):
            off = pl.multiple_of(base + i * chunk, 8)
            pltpu.sync_copy(dest_hbm.at[pl.ds(off, chunk)], idx_v)
            pltpu.sync_copy(x_hbm.at[pl.ds(off, chunk)], rows_v)
            pltpu.sync_copy(zw_hbm.at[pl.ds(off, chunk)], zrows_v)
            cp = pltpu.async_copy(rows_v, xg_hbm.at[idx_v], sem)
            zcp = pltpu.async_copy(zrows_v, zg_hbm.at[idx_v], zsem)
            cp.wait()
            zcp.wait()
            return carry

        lax.fori_loop(0, n_chunks, body, 0)

    return scatter


@functools.lru_cache(maxsize=None)
def _make_gather_rows(n_rows_out, chunk):
    rows_per_w = n_rows_out // _NW
    n_chunks = rows_per_w // chunk
    mesh = plsc.VectorSubcoreMesh(core_axis_name="c", subcore_axis_name="s")

    @functools.partial(
        pl.kernel,
        mesh=mesh,
        out_type=jax.ShapeDtypeStruct((n_rows_out, _D), jnp.float32),
        scratch_types=[
            pltpu.VMEM((chunk,), jnp.int32),
            pltpu.VMEM((chunk, _D), jnp.float32),
            pltpu.SemaphoreType.DMA,
        ],
    )
    def gather(table_hbm, idx_hbm, out_hbm, idx_v, rows_v, sem):
        base = _sc_worker_id() * rows_per_w

        def body(i, carry):
            off = pl.multiple_of(base + i * chunk, 8)
            pltpu.sync_copy(idx_hbm.at[pl.ds(off, chunk)], idx_v)
            pltpu.async_copy(table_hbm.at[idx_v], rows_v, sem).wait()
            pltpu.sync_copy(rows_v, out_hbm.at[pl.ds(off, chunk)])
            return carry

        lax.fori_loop(0, n_chunks, body, 0)

    return gather


def _moe_block_body(be_ref, xg_ref, zg_ref, w1_ref, b1_ref, w2_ref, b2_ref,
                    out_ref):
    xr = xg_ref[...]
    a = jnp.maximum(xr, 0.0)
    h = lax.dot_general(a, w1_ref[0], (((1,), (1,)), ((), ())),
                        preferred_element_type=jnp.float32)
    h = jnp.maximum(h + b1_ref[0, 0, :][None, :], 0.0)
    o = lax.dot_general(h, w2_ref[0], (((1,), (1,)), ((), ())),
                        preferred_element_type=jnp.float32)
    o = o + b2_ref[0, 0, :][None, :]
    out_ref[...] = xr + zg_ref[:, 0:1] * o


def _grouped_fc(be, xg, zg, W1, b1, W2, b2):
    grid_spec = pltpu.PrefetchScalarGridSpec(
        num_scalar_prefetch=1,
        grid=(_G,),
        in_specs=[
            pl.BlockSpec((_B, _D), lambda g, be_r: (g, 0)),
            pl.BlockSpec((_B, _ZW), lambda g, be_r: (g, 0)),
            pl.BlockSpec((1, _S, _D), lambda g, be_r: (be_r[g], 0, 0)),
            pl.BlockSpec((1, 1, _S), lambda g, be_r: (be_r[g], 0, 0)),
            pl.BlockSpec((1, _D, _S), lambda g, be_r: (be_r[g], 0, 0)),
            pl.BlockSpec((1, 1, _D), lambda g, be_r: (be_r[g], 0, 0)),
        ],
        out_specs=pl.BlockSpec((_B, _D), lambda g, be_r: (g, 0)),
    )
    return pl.pallas_call(
        _moe_block_body,
        grid_spec=grid_spec,
        out_shape=jax.ShapeDtypeStruct((_M, _D), jnp.float32),
    )(be, xg, zg, W1, b1.reshape(_E, 1, _S), W2, b2.reshape(_E, 1, _D))


def _scatter_x_z(x, z, dest):
    zwide = jnp.broadcast_to(z, (_N, _ZW))
    return _make_scatter_x_z(64)(x, zwide, dest)


def _gather_out(outg, dest):
    return _make_gather_rows(_N, 64)(outg, dest)


def _routing(yi):
    onehot = (yi[:, None] == jnp.arange(_E, dtype=jnp.int32)[None, :])
    rank_incl = jnp.cumsum(onehot.astype(jnp.int32), axis=0)
    counts = rank_incl[-1]
    rank = jnp.sum(rank_incl * onehot, axis=1) - 1
    nblk = (counts + _B - 1) // _B
    blk_cum = jnp.cumsum(nblk)
    slot_start = (blk_cum - nblk) * _B
    dest = slot_start[yi] + rank
    g_idx = jnp.arange(_G, dtype=jnp.int32)
    be = jnp.sum(
        (g_idx[:, None] >= blk_cum[None, :]).astype(jnp.int32), axis=1)
    be = jnp.minimum(be, _E - 1)
    return dest, be


def kernel(x, y_index, y_hard, z, W1, b1, W2, b2):
    del y_hard
    yi = y_index[:, 0].astype(jnp.int32)
    dest, be = _routing(yi)
    xg, zg = _scatter_x_z(x, z, dest)
    outg = _grouped_fc(be, xg, zg, W1, b1, W2, b2)
    return _gather_out(outg, dest)


# speedup vs baseline: 1.2936x; 1.0416x over previous
import functools

import jax
import jax.numpy as jnp
from jax import lax
from jax.experimental import pallas as pl
from jax.experimental.pallas import tpu as pltpu
from jax.experimental.pallas import tpu_sc as plsc

_N, _D, _S, _E = 8192, 1024, 512, 8
_B = 512
_M = _N + _E * _B
_G = _M // _B
_NW = 32
_ZW = 128


def _sc_worker_id():
    return lax.axis_index("s") * 2 + lax.axis_index("c")


@functools.lru_cache(maxsize=None)
def _make_scatter_x_z(chunk):
    rows_per_w = _N // _NW
    n_chunks = rows_per_w // chunk
    mesh = plsc.VectorSubcoreMesh(core_axis_name="c", subcore_axis_name="s")

    @functools.partial(
        pl.kernel,
        mesh=mesh,
        out_type=[
            jax.ShapeDtypeStruct((_M, _D), jnp.float32),
            jax.ShapeDtypeStruct((_M, _ZW), jnp.float32),
        ],
        scratch_types=[
            pltpu.VMEM((chunk,), jnp.int32),
            pltpu.VMEM((chunk, _D), jnp.float32),
            pltpu.VMEM((chunk, _ZW), jnp.float32),
            pltpu.SemaphoreType.DMA,
            pltpu.SemaphoreType.DMA,
        ],
    )
    def scatter(x_hbm, zw_hbm, dest_hbm, xg_hbm, zg_hbm,
                idx_v, rows_v, zrows_v, sem, zsem):
        base = _sc_worker_id() * rows_per_w

        def body(i, carry):
            off = pl.multiple_of(base + i * chunk, 8)
            pltpu.sync_copy(dest_hbm.at[pl.ds(off, chunk)], idx_v)
            pltpu.sync_copy(x_hbm.at[pl.ds(off, chunk)], rows_v)
            pltpu.sync_copy(zw_hbm.at[pl.ds(off, chunk)], zrows_v)
            cp = pltpu.async_copy(rows_v, xg_hbm.at[idx_v], sem)
            zcp = pltpu.async_copy(zrows_v, zg_hbm.at[idx_v], zsem)
            cp.wait()
            zcp.wait()
            return carry

        lax.fori_loop(0, n_chunks, body, 0)

    return scatter


@functools.lru_cache(maxsize=None)
def _make_gather_rows(n_rows_out, chunk):
    rows_per_w = n_rows_out // _NW
    n_chunks = rows_per_w // chunk
    mesh = plsc.VectorSubcoreMesh(core_axis_name="c", subcore_axis_name="s")

    @functools.partial(
        pl.kernel,
        mesh=mesh,
        out_type=jax.ShapeDtypeStruct((n_rows_out, _D), jnp.float32),
        scratch_types=[
            pltpu.VMEM((chunk,), jnp.int32),
            pltpu.VMEM((chunk, _D), jnp.float32),
            pltpu.SemaphoreType.DMA,
        ],
    )
    def gather(table_hbm, idx_hbm, out_hbm, idx_v, rows_v, sem):
        base = _sc_worker_id() * rows_per_w

        def body(i, carry):
            off = pl.multiple_of(base + i * chunk, 8)
            pltpu.sync_copy(idx_hbm.at[pl.ds(off, chunk)], idx_v)
            pltpu.async_copy(table_hbm.at[idx_v], rows_v, sem).wait()
            pltpu.sync_copy(rows_v, out_hbm.at[pl.ds(off, chunk)])
            return carry

        lax.fori_loop(0, n_chunks, body, 0)

    return gather


def _moe_block_body(be_ref, xg_ref, zg_ref, w1_ref, b1_ref, w2_ref, b2_ref,
                    out_ref):
    xr = xg_ref[...]
    a = jnp.maximum(xr, 0.0)
    h = lax.dot_general(a, w1_ref[0], (((1,), (1,)), ((), ())),
                        preferred_element_type=jnp.float32)
    h = jnp.maximum(h + b1_ref[0, 0, :][None, :], 0.0)
    o = lax.dot_general(h, w2_ref[0], (((1,), (1,)), ((), ())),
                        preferred_element_type=jnp.float32)
    o = o + b2_ref[0, 0, :][None, :]
    out_ref[...] = xr + zg_ref[:, 0:1] * o


def _grouped_fc(be, xg, zg, W1, b1, W2, b2):
    grid_spec = pltpu.PrefetchScalarGridSpec(
        num_scalar_prefetch=1,
        grid=(_G,),
        in_specs=[
            pl.BlockSpec((_B, _D), lambda g, be_r: (g, 0)),
            pl.BlockSpec((_B, _ZW), lambda g, be_r: (g, 0)),
            pl.BlockSpec((1, _S, _D), lambda g, be_r: (be_r[g], 0, 0)),
            pl.BlockSpec((1, 1, _S), lambda g, be_r: (be_r[g], 0, 0)),
            pl.BlockSpec((1, _D, _S), lambda g, be_r: (be_r[g], 0, 0)),
            pl.BlockSpec((1, 1, _D), lambda g, be_r: (be_r[g], 0, 0)),
        ],
        out_specs=pl.BlockSpec((_B, _D), lambda g, be_r: (g, 0)),
    )
    return pl.pallas_call(
        _moe_block_body,
        grid_spec=grid_spec,
        out_shape=jax.ShapeDtypeStruct((_M, _D), jnp.float32),
    )(be, xg, zg, W1, b1.reshape(_E, 1, _S), W2, b2.reshape(_E, 1, _D))


def _scatter_x_z(x, z, dest):
    zwide = jnp.broadcast_to(z, (_N, _ZW))
    return _make_scatter_x_z(64)(x, zwide, dest)


def _gather_out(outg, dest):
    return _make_gather_rows(_N, 64)(outg, dest)


def _routing(yi):
    onehot = (yi[:, None] == jnp.arange(_E, dtype=jnp.int32)[None, :])
    rank_incl = jnp.cumsum(onehot.astype(jnp.int32), axis=0)
    counts = rank_incl[-1]
    rank = jnp.sum(rank_incl * onehot, axis=1) - 1
    nblk = (counts + _B - 1) // _B
    blk_cum = jnp.cumsum(nblk)
    slot_start = (blk_cum - nblk) * _B
    dest = slot_start[yi] + rank
    g_idx = jnp.arange(_G, dtype=jnp.int32)
    be = jnp.sum(
        (g_idx[:, None] >= blk_cum[None, :]).astype(jnp.int32), axis=1)
    be = jnp.minimum(be, _E - 1)
    return dest, be


def kernel(x, y_index, y_hard, z, W1, b1, W2, b2):
    del y_hard
    yi = y_index[:, 0].astype(jnp.int32)
    dest, be = _routing(yi)
    xg, zg = _scatter_x_z(x, z, dest)
    outg = _grouped_fc(be, xg, zg, W1, b1, W2, b2)
    return _gather_out(outg, dest)


# speedup vs baseline: 1.3009x; 1.0057x over previous
import functools

import jax
import jax.numpy as jnp
from jax import lax
from jax.experimental import pallas as pl
from jax.experimental.pallas import tpu as pltpu
from jax.experimental.pallas import tpu_sc as plsc

_N, _D, _S, _E = 8192, 1024, 512, 8
_B = 512
_M = _N + _E * _B
_G = _M // _B
_NW = 32
_ZW = 128


def _sc_worker_id():
    return lax.axis_index("s") * 2 + lax.axis_index("c")


@functools.lru_cache(maxsize=None)
def _make_scatter_x_z(chunk):
    rows_per_w = _N // _NW
    n_chunks = rows_per_w // chunk
    mesh = plsc.VectorSubcoreMesh(core_axis_name="c", subcore_axis_name="s")

    @functools.partial(
        pl.kernel,
        mesh=mesh,
        out_type=[
            jax.ShapeDtypeStruct((_M, _D), jnp.float32),
            jax.ShapeDtypeStruct((_M, _ZW), jnp.float32),
        ],
        scratch_types=[
            pltpu.VMEM((chunk,), jnp.int32),
            pltpu.VMEM((chunk, _D), jnp.float32),
            pltpu.VMEM((chunk, _ZW), jnp.float32),
            pltpu.SemaphoreType.DMA,
            pltpu.SemaphoreType.DMA,
        ],
    )
    def scatter(x_hbm, zw_hbm, dest_hbm, xg_hbm, zg_hbm,
                idx_v, rows_v, zrows_v, sem, zsem):
        base = _sc_worker_id() * rows_per_w

        def body(i, carry):
            off = pl.multiple_of(base + i * chunk, 8)
            pltpu.sync_copy(dest_hbm.at[pl.ds(off, chunk)], idx_v)
            pltpu.sync_copy(x_hbm.at[pl.ds(off, chunk)], rows_v)
            pltpu.sync_copy(zw_hbm.at[pl.ds(off, chunk)], zrows_v)
            cp = pltpu.async_copy(rows_v, xg_hbm.at[idx_v], sem)
            zcp = pltpu.async_copy(zrows_v, zg_hbm.at[idx_v], zsem)
            cp.wait()
            zcp.wait()
            return carry

        lax.fori_loop(0, n_chunks, body, 0)

    return scatter


@functools.lru_cache(maxsize=None)
def _make_gather_rows(n_rows_out, chunk):
    rows_per_w = n_rows_out // _NW
    n_chunks = rows_per_w // chunk
    mesh = plsc.VectorSubcoreMesh(core_axis_name="c", subcore_axis_name="s")

    @functools.partial(
        pl.kernel,
        mesh=mesh,
        out_type=jax.ShapeDtypeStruct((n_rows_out, _D), jnp.float32),
        scratch_types=[
            pltpu.VMEM((chunk,), jnp.int32),
            pltpu.VMEM((chunk, _D), jnp.float32),
            pltpu.SemaphoreType.DMA,
        ],
    )
    def gather(table_hbm, idx_hbm, out_hbm, idx_v, rows_v, sem):
        base = _sc_worker_id() * rows_per_w

        def body(i, carry):
            off = pl.multiple_of(base + i * chunk, 8)
            pltpu.sync_copy(idx_hbm.at[pl.ds(off, chunk)], idx_v)
            pltpu.async_copy(table_hbm.at[idx_v], rows_v, sem).wait()
            pltpu.sync_copy(rows_v, out_hbm.at[pl.ds(off, chunk)])
            return carry

        lax.fori_loop(0, n_chunks, body, 0)

    return gather


def _moe_block_body(be_ref, xg_ref, zg_ref, w1_ref, b1_ref, w2_ref, b2_ref,
                    out_ref):
    g = pl.program_id(0)

    @pl.when(be_ref[g] >= 0)
    def _():
        _moe_block_compute(xg_ref, zg_ref, w1_ref, b1_ref, w2_ref, b2_ref,
                           out_ref)


def _moe_block_compute(xg_ref, zg_ref, w1_ref, b1_ref, w2_ref, b2_ref,
                       out_ref):
    xr = xg_ref[...]
    a = jnp.maximum(xr, 0.0)
    h = lax.dot_general(a, w1_ref[0], (((1,), (1,)), ((), ())),
                        preferred_element_type=jnp.float32)
    h = jnp.maximum(h + b1_ref[0, 0, :][None, :], 0.0)
    o = lax.dot_general(h, w2_ref[0], (((1,), (1,)), ((), ())),
                        preferred_element_type=jnp.float32)
    o = o + b2_ref[0, 0, :][None, :]
    out_ref[...] = xr + zg_ref[:, 0:1] * o


def _grouped_fc(be, xg, zg, W1, b1, W2, b2):
    grid_spec = pltpu.PrefetchScalarGridSpec(
        num_scalar_prefetch=1,
        grid=(_G,),
        in_specs=[
            pl.BlockSpec((_B, _D), lambda g, be_r: (g, 0)),
            pl.BlockSpec((_B, _ZW), lambda g, be_r: (g, 0)),
            pl.BlockSpec((1, _S, _D),
                         lambda g, be_r: (jnp.maximum(be_r[g], 0), 0, 0)),
            pl.BlockSpec((1, 1, _S),
                         lambda g, be_r: (jnp.maximum(be_r[g], 0), 0, 0)),
            pl.BlockSpec((1, _D, _S),
                         lambda g, be_r: (jnp.maximum(be_r[g], 0), 0, 0)),
            pl.BlockSpec((1, 1, _D),
                         lambda g, be_r: (jnp.maximum(be_r[g], 0), 0, 0)),
        ],
        out_specs=pl.BlockSpec((_B, _D), lambda g, be_r: (g, 0)),
    )
    return pl.pallas_call(
        _moe_block_body,
        grid_spec=grid_spec,
        out_shape=jax.ShapeDtypeStruct((_M, _D), jnp.float32),
    )(be, xg, zg, W1, b1.reshape(_E, 1, _S), W2, b2.reshape(_E, 1, _D))


def _scatter_x_z(x, z, dest):
    zwide = jnp.broadcast_to(z, (_N, _ZW))
    return _make_scatter_x_z(64)(x, zwide, dest)


def _gather_out(outg, dest):
    return _make_gather_rows(_N, 64)(outg, dest)


def _routing(yi):
    onehot = (yi[:, None] == jnp.arange(_E, dtype=jnp.int32)[None, :])
    rank_incl = jnp.cumsum(onehot.astype(jnp.int32), axis=0)
    counts = rank_incl[-1]
    rank = jnp.sum(rank_incl * onehot, axis=1) - 1
    nblk = (counts + _B - 1) // _B
    blk_cum = jnp.cumsum(nblk)
    slot_start = (blk_cum - nblk) * _B
    dest = slot_start[yi] + rank
    g_idx = jnp.arange(_G, dtype=jnp.int32)
    be = jnp.sum(
        (g_idx[:, None] >= blk_cum[None, :]).astype(jnp.int32), axis=1)
    be = jnp.where(g_idx < blk_cum[-1], jnp.minimum(be, _E - 1), -1)
    return dest, be


def kernel(x, y_index, y_hard, z, W1, b1, W2, b2):
    del y_hard
    yi = y_index[:, 0].astype(jnp.int32)
    dest, be = _routing(yi)
    xg, zg = _scatter_x_z(x, z, dest)
    outg = _grouped_fc(be, xg, zg, W1, b1, W2, b2)
    return _gather_out(outg, dest)
